# baseline (speedup 1.0000x reference)
import functools

import jax
import jax.numpy as jnp
from jax import lax
from jax.experimental import pallas as pl
from jax.experimental.pallas import tpu as pltpu
from jax.experimental.pallas import tpu_sc as plsc

_NUM_EXPERTS = 64
_TOP_K = 8
_RHO = 0.5
_NUM_NULL = 64
_TB = 512
_NW = 32
_N_TOKENS = 8192
_TPW = _N_TOKENS // _NW


def _tc_logits_kernel(x_ref, wt_ref, b_ref, null_ref,
                      logits_ref, accP_ref, accS_ref):
    t = pl.program_id(0)
    logits = jnp.dot(x_ref[...], wt_ref[...],
                     preferred_element_type=jnp.float32) + b_ref[...]
    logits_ref[...] = logits
    null = null_ref[0, 0]
    m = jnp.maximum(jnp.max(logits, axis=1, keepdims=True), null)
    e = jnp.exp(logits - m)
    s_real = jnp.sum(e, axis=1, keepdims=True)
    z = s_real + _NUM_NULL * jnp.exp(null - m)
    lse = m + jnp.log(z)
    lane = jax.lax.broadcasted_iota(jnp.int32, (1, _NUM_EXPERTS), 1)

    @pl.when(t == 0)
    def _init():
        accP_ref[...] = jnp.zeros_like(accP_ref)
        accS_ref[...] = jnp.zeros_like(accS_ref)

    accP_ref[...] += jnp.sum(e / s_real, axis=0, keepdims=True)
    accS_ref[...] += jnp.where(lane == 0, jnp.sum(lse * lse), 0.0)


def _merge16(ka, va, kb, vb):
    kr = lax.rev(kb, (0,))
    vr = lax.rev(vb, (0,))
    ta = ka >= kr
    kc = jnp.where(ta, ka, kr)
    vc = jnp.where(ta, va, vr)
    return plsc.sort_key_val(kc, vc, descending=True)


def _sc_route_body(logits_hbm, null_hbm, idx_hbm, w_hbm, isn_hbm, cnt_hbm,
                   lv, nullv, idxb, wb, isnb, cnt):
    c = lax.axis_index("c")
    s = lax.axis_index("s")
    wid = s * 2 + c
    pltpu.sync_copy(logits_hbm.at[pl.ds(wid * _TPW * 64, _TPW * 64)], lv)
    pltpu.sync_copy(null_hbm, nullv)
    null_v = nullv[...]
    lane = lax.iota(jnp.int32, 16)
    zeros16 = jnp.zeros((16,), jnp.float32)
    for i in range(4):
        cnt[pl.ds(i * 16, 16)] = zeros16
    valid = lane < _TOP_K
    ones16 = jnp.ones((16,), jnp.float32)

    lane0 = jnp.zeros((16,), jnp.int32)

    @plsc.parallel_loop(0, _TPW, 1, unroll=8)
    def _token_body(t):
        base = t * 64
        k0 = lv[pl.ds(base, 16)]
        k1 = lv[pl.ds(base + 16, 16)]
        k2 = lv[pl.ds(base + 32, 16)]
        k3 = lv[pl.ds(base + 48, 16)]

        s0 = plsc.sort_key_val(k0, lane, descending=True)
        s1 = plsc.sort_key_val(k1, lane + 16, descending=True)
        s2 = plsc.sort_key_val(k2, lane + 32, descending=True)
        s3 = plsc.sort_key_val(k3, lane + 48, descending=True)
        mk, mvv = _merge16(s0[0], s0[1], s1[0], s1[1])
        nk, nvv = _merge16(s2[0], s2[1], s3[0], s3[1])
        fk, fv = _merge16(mk, mvv, nk, nvv)

        m_bcast = lax.gather(
            fk, lane0[:, None],
            dimension_numbers=lax.GatherDimensionNumbers(
                offset_dims=(), collapsed_slice_dims=(0,),
                start_index_map=(0,)),
            slice_sizes=(1,),
            mode=lax.GatherScatterMode.PROMISE_IN_BOUNDS)
        m_v = jnp.maximum(m_bcast, null_v)
        e_sum = (jnp.exp(k0 - m_v) + jnp.exp(k1 - m_v)
                 + jnp.exp(k2 - m_v) + jnp.exp(k3 - m_v))
        s_real = jnp.sum(e_sum, axis=0)
        z_v = s_real + 64.0 * jnp.exp(null_v - m_v)

        isnull = (fk < null_v) & valid
        real = valid & (~isnull)
        nreal = plsc.all_reduce_population_count(real)
        out_idx = jnp.where(isnull, _NUM_EXPERTS + lane - nreal, fv)
        w_pre = jnp.exp(fk - m_v) / z_v
        w_real = jnp.where(real, w_pre, 0.0)
        wsum = jnp.sum(w_real, axis=0)
        w_out = w_real / jnp.maximum(wsum, 1e-6)

        off = t * _TOP_K
        plsc.store_compressed(idxb.at[pl.ds(off, 16)], out_idx, mask=valid)
        plsc.store_compressed(wb.at[pl.ds(off, 16)], w_out, mask=valid)
        plsc.store_compressed(isnb.at[pl.ds(off, 16)],
                              isnull.astype(jnp.int32), mask=valid)
        plsc.addupdate_scatter(cnt, [fv], ones16, mask=real)

    n_out = _TPW * _TOP_K
    pltpu.sync_copy(idxb.at[pl.ds(0, n_out)],
                    idx_hbm.at[pl.ds(wid * n_out, n_out)])
    pltpu.sync_copy(wb.at[pl.ds(0, n_out)],
                    w_hbm.at[pl.ds(wid * n_out, n_out)])
    pltpu.sync_copy(isnb.at[pl.ds(0, n_out)],
                    isn_hbm.at[pl.ds(wid * n_out, n_out)])
    pltpu.sync_copy(cnt, cnt_hbm.at[pl.ds(wid * 64, 64)])


_sc_route = functools.partial(
    pl.kernel,
    out_type=(
        jax.ShapeDtypeStruct((_N_TOKENS * _TOP_K,), jnp.int32),
        jax.ShapeDtypeStruct((_N_TOKENS * _TOP_K,), jnp.float32),
        jax.ShapeDtypeStruct((_N_TOKENS * _TOP_K,), jnp.int32),
        jax.ShapeDtypeStruct((_NW * 64,), jnp.float32),
    ),
    mesh=plsc.VectorSubcoreMesh(core_axis_name="c", subcore_axis_name="s",
                                num_cores=2, num_subcores=16),
    compiler_params=pltpu.CompilerParams(needs_layout_passes=False),
    scratch_types=[
        pltpu.VMEM((_TPW * 64,), jnp.float32),
        pltpu.VMEM((16,), jnp.float32),
        pltpu.VMEM((_TPW * _TOP_K + 8,), jnp.int32),
        pltpu.VMEM((_TPW * _TOP_K + 8,), jnp.float32),
        pltpu.VMEM((_TPW * _TOP_K + 8,), jnp.int32),
        pltpu.VMEM((64,), jnp.float32),
    ],
)(_sc_route_body)


def _combine_kernel(accP_ref, accS_ref, cnt_ref, aux_ref):
    counts = jnp.sum(cnt_ref[...], axis=0, keepdims=True)
    csum = jnp.sum(counts)
    total = jnp.maximum(csum, 1e-6)
    p_real = accP_ref[...] / _N_TOKENS
    l_bal = _NUM_EXPERTS * jnp.sum((counts / total) * p_real)
    lane = jax.lax.broadcasted_iota(jnp.int32, (1, _NUM_EXPERTS), 1)
    l_z = jnp.sum(jnp.where(lane == 0, accS_ref[...], 0.0)) / _N_TOKENS
    null_rate = (_N_TOKENS * _TOP_K - csum) / (_N_TOKENS * _TOP_K)
    l_null = (null_rate - _RHO) ** 2
    aux = 0.02 * l_bal + 0.001 * l_z + 0.01 * l_null
    aux_ref[...] = jnp.broadcast_to(aux, (1, 1))


@jax.jit
def kernel(x, W, logit_bias, null_logit):
    B, T, D = x.shape
    xf = x.reshape(_N_TOKENS, D)
    wt = W.T
    bias = logit_bias.reshape(1, _NUM_EXPERTS)
    null11 = jnp.reshape(null_logit, (1, 1)).astype(jnp.float32)
    null16 = jnp.broadcast_to(null_logit.astype(jnp.float32), (16,))

    n_blocks = _N_TOKENS // _TB
    tok_spec = lambda w: pl.BlockSpec((_TB, w), lambda i: (i, 0))
    fix_spec = lambda s: pl.BlockSpec(s, lambda i: (0, 0))
    logits, accP, accS = pl.pallas_call(
        _tc_logits_kernel,
        grid=(n_blocks,),
        in_specs=[tok_spec(D), fix_spec((D, _NUM_EXPERTS)),
                  fix_spec((1, _NUM_EXPERTS)), fix_spec((1, 1))],
        out_specs=(tok_spec(_NUM_EXPERTS), fix_spec((1, _NUM_EXPERTS)),
                   fix_spec((1, _NUM_EXPERTS))),
        out_shape=(
            jax.ShapeDtypeStruct((_N_TOKENS, _NUM_EXPERTS), jnp.float32),
            jax.ShapeDtypeStruct((1, _NUM_EXPERTS), jnp.float32),
            jax.ShapeDtypeStruct((1, _NUM_EXPERTS), jnp.float32),
        ),
    )(xf, wt, bias, null11)

    idxf, wf, isnf, cnt = _sc_route(logits.reshape(-1), null16)

    aux = pl.pallas_call(
        _combine_kernel,
        grid=(1,),
        in_specs=[fix_spec((1, _NUM_EXPERTS)), fix_spec((1, _NUM_EXPERTS)),
                  pl.BlockSpec((_NW, 64), lambda i: (0, 0))],
        out_specs=fix_spec((1, 1)),
        out_shape=jax.ShapeDtypeStruct((1, 1), jnp.float32),
    )(accP, accS, cnt.reshape(_NW, 64))

    return (idxf.reshape(B, T, _TOP_K),
            wf.reshape(B, T, _TOP_K),
            (isnf != 0).reshape(B, T, _TOP_K),
            aux[0, 0])


# speedup vs baseline: 1.0601x; 1.0601x over previous
import functools

import jax
import jax.numpy as jnp
from jax import lax
from jax.experimental import pallas as pl
from jax.experimental.pallas import tpu as pltpu
from jax.experimental.pallas import tpu_sc as plsc

_NUM_EXPERTS = 64
_TOP_K = 8
_RHO = 0.5
_NUM_NULL = 64
_TB = 1024
_NW = 32
_N_TOKENS = 8192
_TPW = _N_TOKENS // _NW


def _tc_logits_kernel(x_ref, wt_ref, b_ref, null_ref,
                      logits_ref, accP_ref, accS_ref):
    t = pl.program_id(0)
    logits = jnp.dot(x_ref[...], wt_ref[...],
                     preferred_element_type=jnp.float32) + b_ref[...]
    logits_ref[...] = logits
    null = null_ref[0, 0]
    m = jnp.maximum(jnp.max(logits, axis=1, keepdims=True), null)
    e = jnp.exp(logits - m)
    s_real = jnp.sum(e, axis=1, keepdims=True)
    z = s_real + _NUM_NULL * jnp.exp(null - m)
    lse = m + jnp.log(z)
    lane = jax.lax.broadcasted_iota(jnp.int32, (1, _NUM_EXPERTS), 1)

    @pl.when(t == 0)
    def _init():
        accP_ref[...] = jnp.zeros_like(accP_ref)
        accS_ref[...] = jnp.zeros_like(accS_ref)

    accP_ref[...] += jnp.sum(e * (1.0 / s_real), axis=0, keepdims=True)
    accS_ref[...] += jnp.where(lane == 0, jnp.sum(lse * lse), 0.0)


def _merge16(ka, va, kb, vb):
    kr = lax.rev(kb, (0,))
    vr = lax.rev(vb, (0,))
    ta = ka >= kr
    kc = jnp.where(ta, ka, kr)
    vc = jnp.where(ta, va, vr)
    return plsc.sort_key_val(kc, vc, descending=True)


def _sc_route_body(logits_hbm, null_hbm, idx_hbm, w_hbm, isn_hbm, cnt_hbm,
                   lv, nullv, idxb, wb, isnb, cnt):
    c = lax.axis_index("c")
    s = lax.axis_index("s")
    wid = s * 2 + c
    pltpu.sync_copy(logits_hbm.at[pl.ds(wid * _TPW * 64, _TPW * 64)], lv)
    pltpu.sync_copy(null_hbm, nullv)
    null_v = nullv[...]
    lane = lax.iota(jnp.int32, 16)
    zeros16 = jnp.zeros((16,), jnp.float32)
    for i in range(4):
        cnt[pl.ds(i * 16, 16)] = zeros16
    valid = lane < _TOP_K
    ones16 = jnp.ones((16,), jnp.float32)

    lane0 = jnp.zeros((16,), jnp.int32)

    @plsc.parallel_loop(0, _TPW, 1, unroll=8)
    def _token_body(t):
        base = t * 64
        k0 = lv[pl.ds(base, 16)]
        k1 = lv[pl.ds(base + 16, 16)]
        k2 = lv[pl.ds(base + 32, 16)]
        k3 = lv[pl.ds(base + 48, 16)]

        s0 = plsc.sort_key_val(k0, lane, descending=True)
        s1 = plsc.sort_key_val(k1, lane + 16, descending=True)
        s2 = plsc.sort_key_val(k2, lane + 32, descending=True)
        s3 = plsc.sort_key_val(k3, lane + 48, descending=True)
        mk, mvv = _merge16(s0[0], s0[1], s1[0], s1[1])
        nk, nvv = _merge16(s2[0], s2[1], s3[0], s3[1])
        fk, fv = _merge16(mk, mvv, nk, nvv)

        m_bcast = lax.gather(
            fk, lane0[:, None],
            dimension_numbers=lax.GatherDimensionNumbers(
                offset_dims=(), collapsed_slice_dims=(0,),
                start_index_map=(0,)),
            slice_sizes=(1,),
            mode=lax.GatherScatterMode.PROMISE_IN_BOUNDS)
        m_v = jnp.maximum(m_bcast, null_v)
        e_sum = (jnp.exp(k0 - m_v) + jnp.exp(k1 - m_v)
                 + jnp.exp(k2 - m_v) + jnp.exp(k3 - m_v))
        s_real = jnp.sum(e_sum, axis=0)
        z_v = s_real + 64.0 * jnp.exp(null_v - m_v)

        isnull = (fk < null_v) & valid
        real = valid & (~isnull)
        nreal = plsc.all_reduce_population_count(real)
        out_idx = jnp.where(isnull, _NUM_EXPERTS + lane - nreal, fv)
        w_pre = jnp.exp(fk - m_v) / z_v
        w_real = jnp.where(real, w_pre, 0.0)
        wsum = jnp.sum(w_real, axis=0)
        w_out = w_real / jnp.maximum(wsum, 1e-6)

        off = t * _TOP_K
        plsc.store_compressed(idxb.at[pl.ds(off, 16)], out_idx, mask=valid)
        plsc.store_compressed(wb.at[pl.ds(off, 16)], w_out, mask=valid)
        plsc.store_compressed(isnb.at[pl.ds(off, 16)],
                              isnull.astype(jnp.int32), mask=valid)
        plsc.addupdate_scatter(cnt, [fv], ones16, mask=real)

    n_out = _TPW * _TOP_K
    pltpu.sync_copy(idxb.at[pl.ds(0, n_out)],
                    idx_hbm.at[pl.ds(wid * n_out, n_out)])
    pltpu.sync_copy(wb.at[pl.ds(0, n_out)],
                    w_hbm.at[pl.ds(wid * n_out, n_out)])
    pltpu.sync_copy(isnb.at[pl.ds(0, n_out)],
                    isn_hbm.at[pl.ds(wid * n_out, n_out)])
    pltpu.sync_copy(cnt, cnt_hbm.at[pl.ds(wid * 64, 64)])


_sc_route = functools.partial(
    pl.kernel,
    out_type=(
        jax.ShapeDtypeStruct((_N_TOKENS * _TOP_K,), jnp.int32),
        jax.ShapeDtypeStruct((_N_TOKENS * _TOP_K,), jnp.float32),
        jax.ShapeDtypeStruct((_N_TOKENS * _TOP_K,), jnp.int32),
        jax.ShapeDtypeStruct((_NW * 64,), jnp.float32),
    ),
    mesh=plsc.VectorSubcoreMesh(core_axis_name="c", subcore_axis_name="s",
                                num_cores=2, num_subcores=16),
    compiler_params=pltpu.CompilerParams(needs_layout_passes=False),
    scratch_types=[
        pltpu.VMEM((_TPW * 64,), jnp.float32),
        pltpu.VMEM((16,), jnp.float32),
        pltpu.VMEM((_TPW * _TOP_K + 8,), jnp.int32),
        pltpu.VMEM((_TPW * _TOP_K + 8,), jnp.float32),
        pltpu.VMEM((_TPW * _TOP_K + 8,), jnp.int32),
        pltpu.VMEM((64,), jnp.float32),
    ],
)(_sc_route_body)


def _combine_kernel(accP_ref, accS_ref, cnt_ref, aux_ref):
    counts = jnp.sum(cnt_ref[...], axis=0, keepdims=True)
    csum = jnp.sum(counts)
    total = jnp.maximum(csum, 1e-6)
    p_real = accP_ref[...] / _N_TOKENS
    l_bal = _NUM_EXPERTS * jnp.sum((counts / total) * p_real)
    lane = jax.lax.broadcasted_iota(jnp.int32, (1, _NUM_EXPERTS), 1)
    l_z = jnp.sum(jnp.where(lane == 0, accS_ref[...], 0.0)) / _N_TOKENS
    null_rate = (_N_TOKENS * _TOP_K - csum) / (_N_TOKENS * _TOP_K)
    l_null = (null_rate - _RHO) ** 2
    aux = 0.02 * l_bal + 0.001 * l_z + 0.01 * l_null
    aux_ref[...] = jnp.broadcast_to(aux, (1, 1))


@jax.jit
def kernel(x, W, logit_bias, null_logit):
    B, T, D = x.shape
    xf = x.reshape(_N_TOKENS, D)
    wt = W.T
    bias = logit_bias.reshape(1, _NUM_EXPERTS)
    null11 = jnp.reshape(null_logit, (1, 1)).astype(jnp.float32)
    null16 = jnp.broadcast_to(null_logit.astype(jnp.float32), (16,))

    n_blocks = _N_TOKENS // _TB
    tok_spec = lambda w: pl.BlockSpec((_TB, w), lambda i: (i, 0))
    fix_spec = lambda s: pl.BlockSpec(s, lambda i: (0, 0))
    logits, accP, accS = pl.pallas_call(
        _tc_logits_kernel,
        grid=(n_blocks,),
        in_specs=[tok_spec(D), fix_spec((D, _NUM_EXPERTS)),
                  fix_spec((1, _NUM_EXPERTS)), fix_spec((1, 1))],
        out_specs=(tok_spec(_NUM_EXPERTS), fix_spec((1, _NUM_EXPERTS)),
                   fix_spec((1, _NUM_EXPERTS))),
        out_shape=(
            jax.ShapeDtypeStruct((_N_TOKENS, _NUM_EXPERTS), jnp.float32),
            jax.ShapeDtypeStruct((1, _NUM_EXPERTS), jnp.float32),
            jax.ShapeDtypeStruct((1, _NUM_EXPERTS), jnp.float32),
        ),
    )(xf, wt, bias, null11)

    idxf, wf, isnf, cnt = _sc_route(logits.reshape(-1), null16)

    aux = pl.pallas_call(
        _combine_kernel,
        grid=(1,),
        in_specs=[fix_spec((1, _NUM_EXPERTS)), fix_spec((1, _NUM_EXPERTS)),
                  pl.BlockSpec((_NW, 64), lambda i: (0, 0))],
        out_specs=fix_spec((1, 1)),
        out_shape=jax.ShapeDtypeStruct((1, 1), jnp.float32),
    )(accP, accS, cnt.reshape(_NW, 64))

    return (idxf.reshape(B, T, _TOP_K),
            wf.reshape(B, T, _TOP_K),
            (isnf != 0).reshape(B, T, _TOP_K),
            aux[0, 0])
